# baseline (speedup 1.0000x reference)
import jax
import jax.numpy as jnp
from jax import lax
from jax.experimental import pallas as pl
from jax.experimental.pallas import tpu as pltpu


def _stats_kernel(x_ref, g_ref, s_ref, *, tb):
    r0 = jnp.maximum(x_ref[0], 0.0)
    g = lax.dot_general(r0, r0, (((1,), (1,)), ((), ())),
                        preferred_element_type=jnp.float32)
    s = jnp.sum(r0, axis=-1, keepdims=True)
    for b in range(1, tb):
        rb = jnp.maximum(x_ref[b], 0.0)
        g = g + lax.dot_general(rb, rb, (((1,), (1,)), ((), ())),
                                preferred_element_type=jnp.float32)
        s = s + jnp.sum(rb, axis=-1, keepdims=True)
    g_ref[0] = g
    s_ref[0] = s


def _fused_kernel(x_ref, w_ref, scale_ref, shift_ref, o_ref, *, h, w, cs):
    r = jnp.maximum(x_ref[0], 0.0)
    c_out = o_ref.shape[1]
    m = r.shape[-1]

    lane = lax.broadcasted_iota(jnp.int32, (1, m), 1)
    col = lane % w
    row = lane // w
    cv = 3 - (col == 0).astype(jnp.int32) - (col == w - 1).astype(jnp.int32)
    rv = 3 - (row == 0).astype(jnp.int32) - (row == h - 1).astype(jnp.int32)
    rcnt = 1.0 / (rv * cv).astype(jnp.float32)
    lmask = col > 0
    rmask = col < w - 1

    zc = jnp.zeros((cs, 1), jnp.float32)
    zr = jnp.zeros((cs, w), jnp.float32)
    for c0 in range(0, c_out, cs):
        y = jnp.dot(w_ref[c0:c0 + cs, :], r,
                    preferred_element_type=jnp.float32)
        left = jnp.concatenate([zc, y[:, :m - 1]], axis=1)
        right = jnp.concatenate([y[:, 1:], zc], axis=1)
        hsum = y + jnp.where(lmask, left, 0.0) + jnp.where(rmask, right, 0.0)
        up = jnp.concatenate([zr, hsum[:, :m - w]], axis=1)
        down = jnp.concatenate([hsum[:, w:], zr], axis=1)
        vsum = hsum + up + down
        o_ref[0, c0:c0 + cs, :] = (vsum * rcnt) * scale_ref[c0:c0 + cs, :] \
            + shift_ref[c0:c0 + cs, :]


def kernel(x, weight, gamma, beta, eps=1e-5):
    n, c_in, h, w = x.shape
    c_out = weight.shape[0]
    hw = h * w
    m_total = n * hw

    x3 = x.astype(jnp.float32).reshape(n, c_in, hw)
    w2 = weight.reshape(c_out, c_in).astype(jnp.float32)

    tb = 8
    while n % tb:
        tb -= 1
    nb = n // tb

    gp, sp = pl.pallas_call(
        lambda xr, gr, sr: _stats_kernel(xr, gr, sr, tb=tb),
        grid=(nb,),
        in_specs=[pl.BlockSpec((tb, c_in, hw), lambda i: (i, 0, 0))],
        out_specs=[
            pl.BlockSpec((1, c_in, c_in), lambda i: (i, 0, 0)),
            pl.BlockSpec((1, c_in, 1), lambda i: (i, 0, 0)),
        ],
        out_shape=[
            jax.ShapeDtypeStruct((nb, c_in, c_in), jnp.float32),
            jax.ShapeDtypeStruct((nb, c_in, 1), jnp.float32),
        ],
        compiler_params=pltpu.CompilerParams(dimension_semantics=("parallel",)),
    )(x3)

    g = jnp.sum(gp, axis=0)
    s = jnp.sum(sp, axis=0)[:, 0]
    mean = (w2 @ s) / m_total
    sumsq = jnp.sum((w2 @ g) * w2, axis=1)
    var = sumsq / m_total - mean * mean
    ch_scale = gamma.astype(jnp.float32) * lax.rsqrt(var + eps)
    ch_shift = beta.astype(jnp.float32) - mean * ch_scale

    out = pl.pallas_call(
        lambda xr, wr, scr, shr, orr: _fused_kernel(
            xr, wr, scr, shr, orr, h=h, w=w, cs=min(32, c_out)),
        grid=(n,),
        in_specs=[
            pl.BlockSpec((1, c_in, hw), lambda i: (i, 0, 0)),
            pl.BlockSpec((c_out, c_in), lambda i: (0, 0)),
            pl.BlockSpec((c_out, 1), lambda i: (0, 0)),
            pl.BlockSpec((c_out, 1), lambda i: (0, 0)),
        ],
        out_specs=pl.BlockSpec((1, c_out, hw), lambda i: (i, 0, 0)),
        out_shape=jax.ShapeDtypeStruct((n, c_out, hw), jnp.float32),
        compiler_params=pltpu.CompilerParams(dimension_semantics=("parallel",)),
    )(x3, w2, ch_scale.reshape(c_out, 1), ch_shift.reshape(c_out, 1))

    return out.reshape(n, c_out, h, w)


# speedup vs baseline: 1.3394x; 1.3394x over previous
import numpy as np

import jax
import jax.numpy as jnp
from jax import lax
from jax.experimental import pallas as pl
from jax.experimental.pallas import tpu as pltpu


def _stats_kernel(x_ref, g_ref, s_ref, *, tb):
    r0 = jnp.maximum(x_ref[0], 0.0)
    g = lax.dot_general(r0, r0, (((1,), (1,)), ((), ())),
                        preferred_element_type=jnp.float32)
    s = jnp.sum(r0, axis=-1, keepdims=True)
    for b in range(1, tb):
        rb = jnp.maximum(x_ref[b], 0.0)
        g = g + lax.dot_general(rb, rb, (((1,), (1,)), ((), ())),
                                preferred_element_type=jnp.float32)
        s = s + jnp.sum(rb, axis=-1, keepdims=True)
    g_ref[0] = g
    s_ref[0] = s


def _fused_kernel(x_ref, w_ref, p_ref, shift_ref, o_ref, *, tb):
    for b in range(tb):
        r = jnp.maximum(x_ref[b], 0.0)
        y = jnp.dot(w_ref[...], r, preferred_element_type=jnp.float32)
        o_ref[b] = jnp.dot(y, p_ref[...],
                           preferred_element_type=jnp.float32) + shift_ref[...]


def _pool_matrix(h, w):
    hw = h * w
    rr = np.arange(hw) // w
    cc = np.arange(hw) % w
    near_r = np.abs(rr[:, None] - rr[None, :]) <= 1
    near_c = np.abs(cc[:, None] - cc[None, :]) <= 1
    band = (near_r & near_c).astype(np.float32)
    return band / band.sum(axis=0, keepdims=True)


def kernel(x, weight, gamma, beta, eps=1e-5):
    n, c_in, h, w = x.shape
    c_out = weight.shape[0]
    hw = h * w
    m_total = n * hw

    x3 = x.astype(jnp.float32).reshape(n, c_in, hw)
    w2 = weight.reshape(c_out, c_in).astype(jnp.float32)

    tb = 8
    while n % tb:
        tb -= 1
    nb = n // tb

    gp, sp = pl.pallas_call(
        lambda xr, gr, sr: _stats_kernel(xr, gr, sr, tb=tb),
        grid=(nb,),
        in_specs=[pl.BlockSpec((tb, c_in, hw), lambda i: (i, 0, 0))],
        out_specs=[
            pl.BlockSpec((1, c_in, c_in), lambda i: (i, 0, 0)),
            pl.BlockSpec((1, c_in, 1), lambda i: (i, 0, 0)),
        ],
        out_shape=[
            jax.ShapeDtypeStruct((nb, c_in, c_in), jnp.float32),
            jax.ShapeDtypeStruct((nb, c_in, 1), jnp.float32),
        ],
        compiler_params=pltpu.CompilerParams(dimension_semantics=("parallel",)),
    )(x3)

    g = jnp.sum(gp, axis=0)
    s = jnp.sum(sp, axis=0)[:, 0]
    mean = (w2 @ s) / m_total
    sumsq = jnp.sum((w2 @ g) * w2, axis=1)
    var = sumsq / m_total - mean * mean
    ch_scale = gamma.astype(jnp.float32) * lax.rsqrt(var + eps)
    ch_shift = beta.astype(jnp.float32) - mean * ch_scale
    w_scaled = ch_scale[:, None] * w2

    pool_mat = jnp.asarray(_pool_matrix(h, w))

    out = pl.pallas_call(
        lambda xr, wr, pr, shr, orr: _fused_kernel(xr, wr, pr, shr, orr, tb=tb),
        grid=(nb,),
        in_specs=[
            pl.BlockSpec((tb, c_in, hw), lambda i: (i, 0, 0)),
            pl.BlockSpec((c_out, c_in), lambda i: (0, 0)),
            pl.BlockSpec((hw, hw), lambda i: (0, 0)),
            pl.BlockSpec((c_out, 1), lambda i: (0, 0)),
        ],
        out_specs=pl.BlockSpec((tb, c_out, hw), lambda i: (i, 0, 0)),
        out_shape=jax.ShapeDtypeStruct((n, c_out, hw), jnp.float32),
        compiler_params=pltpu.CompilerParams(dimension_semantics=("parallel",)),
    )(x3, w_scaled, pool_mat, ch_shift.reshape(c_out, 1))

    return out.reshape(n, c_out, h, w)
